# Initial kernel scaffold; baseline (speedup 1.0000x reference)
import jax
import jax.numpy as jnp
from jax.experimental import pallas as pl


def kernel(features, vertices, edges, faces, W1m, b1m, W2m, b2m, Wemb, bemb, Wself, Wnbr, bconv):
    raise NotImplementedError("write your pallas kernel here")


# speedup vs baseline: 4.8925x; 4.8925x over previous
import functools

import numpy as np
import jax
import jax.numpy as jnp
from jax import lax
from jax.experimental import pallas as pl
from jax.experimental.pallas import tpu as pltpu
from jax.experimental.pallas import tpu_sc as plsc

N = 10000
E = 160000
D = 128
MH = 32
ED = 8
TRI = ED * (ED + 1) // 2
NL = 10

NC = 2
NS = 16
NW = NC * NS
K = 128
NCH = 40
EW = K * NCH
E_PAD = NW * EW
N_PAD = 10240
STRIPE = N_PAD // NS
EB = 4096

_MESH = plsc.VectorSubcoreMesh(
    core_axis_name="c", subcore_axis_name="s", num_cores=NC, num_subcores=NS
)


def _ev_body(vert, srcr, dstr, ev_out, srcv, dstv, ra, rb, sem_a, sem_b):
    c = lax.axis_index("c")
    s = lax.axis_index("s")
    base0 = (c * NS + s) * EW

    def chunk(g, carry):
        base = base0 + g * K
        pltpu.sync_copy(srcr.at[pl.ds(base, K)], srcv)
        pltpu.sync_copy(dstr.at[pl.ds(base, K)], dstv)
        cpa = pltpu.async_copy(vert.at[srcv], ra, sem_a)
        cpb = pltpu.async_copy(vert.at[dstv], rb, sem_b)
        cpa.wait()
        cpb.wait()

        def sub(i, carry2):
            ra[i, pl.ds(0, 16)] = ra[i, pl.ds(0, 16)] - rb[i, pl.ds(0, 16)]
            return carry2

        lax.fori_loop(0, K, sub, 0)
        pltpu.sync_copy(ra, ev_out.at[pl.ds(base, K)])
        return carry

    lax.fori_loop(0, NCH, chunk, 0)


_ev_call = functools.partial(
    pl.kernel,
    out_type=jax.ShapeDtypeStruct((E_PAD, D), jnp.float32),
    mesh=_MESH,
    scratch_types=[
        pltpu.VMEM((K,), jnp.int32),
        pltpu.VMEM((K,), jnp.int32),
        pltpu.VMEM((K, D), jnp.float32),
        pltpu.VMEM((K, D), jnp.float32),
        pltpu.SemaphoreType.DMA,
        pltpu.SemaphoreType.DMA,
    ],
)(_ev_body)


def _wedge_body(ev_ref, w1_ref, b1_ref, w2_ref, b2_ref, we_ref, be_ref,
                g1_ref, g2_ref, out_ref):
    ev = ev_ref[...][:, :16]
    pid = pl.program_id(0)
    eidx = lax.broadcasted_iota(jnp.int32, (EB, 1), 0) + pid * EB
    mask = (eidx < E).astype(jnp.float32)
    g1 = g1_ref[...]
    g2 = g2_ref[...]
    cols = []
    for li in range(NL):
        h = jnp.maximum(
            jnp.dot(ev, w1_ref[li], preferred_element_type=jnp.float32)
            + b1_ref[li], 0.0)
        m = jnp.dot(h, w2_ref[li], preferred_element_type=jnp.float32) + b2_ref[li]
        emb = jnp.dot(ev, we_ref[li], preferred_element_type=jnp.float32) + be_ref[li]
        embi = jnp.dot(emb, g1, preferred_element_type=jnp.float32)
        t = jnp.dot(embi * m, g2, preferred_element_type=jnp.float32)
        q = jnp.sum(t * t, axis=1, keepdims=True)
        cols.append(jnp.exp(-q) * mask)
    cols.append(jnp.zeros((EB, D - NL), jnp.float32))
    out_ref[...] = jnp.concatenate(cols, axis=1)


def _wedge_call(evp, w1p, b1, w2, b2, wep, be, g1, g2):
    n_blocks = E_PAD // EB
    zero3 = lambda i: (0, 0, 0)
    zero2 = lambda i: (0, 0)
    return pl.pallas_call(
        _wedge_body,
        grid=(n_blocks,),
        in_specs=[
            pl.BlockSpec((EB, D), lambda i: (i, 0)),
            pl.BlockSpec((NL, 16, MH), zero3),
            pl.BlockSpec((NL, MH), zero2),
            pl.BlockSpec((NL, MH, TRI), zero3),
            pl.BlockSpec((NL, TRI), zero2),
            pl.BlockSpec((NL, 16, ED), zero3),
            pl.BlockSpec((NL, ED), zero2),
            pl.BlockSpec((ED, TRI), zero2),
            pl.BlockSpec((TRI, ED), zero2),
        ],
        out_specs=pl.BlockSpec((EB, D), lambda i: (i, 0)),
        out_shape=jax.ShapeDtypeStruct((E_PAD, D), jnp.float32),
    )(evp, w1p, b1, w2, b2, wep, be, g1, g2)


def _wsum_body(w_hbm, dst_hbm, ws_out, dstv, wrows, ws_s, sem):
    c = lax.axis_index("c")
    s = lax.axis_index("s")

    def zrows(i, carry):
        for j in range(D // 16):
            wrows[i, pl.ds(j * 16, 16)] = jnp.zeros((16,), jnp.float32)
        return carry

    lax.fori_loop(0, K, zrows, 0)
    for j in range(STRIPE // K):
        pltpu.sync_copy(wrows, ws_s.at[pl.ds(s * STRIPE + j * K, K)])
    plsc.subcore_barrier()

    base0 = (c * NS + s) * EW

    def chunk(g, carry):
        base = base0 + g * K
        pltpu.sync_copy(dst_hbm.at[pl.ds(base, K)], dstv)
        pltpu.sync_copy(w_hbm.at[pl.ds(base, K)], wrows)
        pltpu.sync_copy(wrows, ws_s.at[dstv], add=True)
        return carry

    lax.fori_loop(0, NCH, chunk, 0)
    plsc.subcore_barrier()
    for j in range(STRIPE // K):
        r0 = s * STRIPE + j * K
        pltpu.sync_copy(ws_s.at[pl.ds(r0, K)], ws_out.at[c, pl.ds(r0, K)])


_wsum_call = functools.partial(
    pl.kernel,
    out_type=jax.ShapeDtypeStruct((NC, N_PAD, D), jnp.float32),
    mesh=_MESH,
    scratch_types=[
        pltpu.VMEM((K,), jnp.int32),
        pltpu.VMEM((K, D), jnp.float32),
        pltpu.VMEM_SHARED((N_PAD, D), jnp.float32),
        pltpu.SemaphoreType.DMA,
    ],
)(_wsum_body)


def _spmm_body(x_hbm, src_hbm, dst_hbm, w_hbm, agg_out,
               srcv, dstv, wv, rows, agg_s, sem):
    c = lax.axis_index("c")
    s = lax.axis_index("s")

    def zrows(i, carry):
        for j in range(D // 16):
            rows[i, pl.ds(j * 16, 16)] = jnp.zeros((16,), jnp.float32)
        return carry

    lax.fori_loop(0, K, zrows, 0)
    for j in range(STRIPE // K):
        pltpu.sync_copy(rows, agg_s.at[pl.ds(s * STRIPE + j * K, K)])
    plsc.subcore_barrier()

    base0 = (c * NS + s) * EW

    def chunk(g, carry):
        base = base0 + g * K
        pltpu.sync_copy(src_hbm.at[pl.ds(base, K)], srcv)
        pltpu.sync_copy(dst_hbm.at[pl.ds(base, K)], dstv)
        pltpu.sync_copy(w_hbm.at[pl.ds(base, K)], wv)
        pltpu.async_copy(x_hbm.at[srcv], rows, sem).wait()

        def scale(g2, carry2):
            wgrp = wv[pl.ds(g2 * 16, 16)]
            for l in range(16):
                wspl = wgrp.at[jnp.full((16,), l, jnp.int32)].get(
                    mode="promise_in_bounds")
                i = g2 * 16 + l
                for j in range(D // 16):
                    sl = pl.ds(j * 16, 16)
                    rows[i, sl] = rows[i, sl] * wspl
            return carry2

        lax.fori_loop(0, K // 16, scale, 0)

        pltpu.sync_copy(rows, agg_s.at[dstv], add=True)
        return carry

    lax.fori_loop(0, NCH, chunk, 0)
    plsc.subcore_barrier()

    for j in range(STRIPE // K):
        r0 = s * STRIPE + j * K
        pltpu.sync_copy(agg_s.at[pl.ds(r0, K)], agg_out.at[c, pl.ds(r0, K)])


_spmm_call = functools.partial(
    pl.kernel,
    out_type=jax.ShapeDtypeStruct((NC, N_PAD, D), jnp.float32),
    mesh=_MESH,
    scratch_types=[
        pltpu.VMEM((K,), jnp.int32),
        pltpu.VMEM((K,), jnp.int32),
        pltpu.VMEM((K,), jnp.float32),
        pltpu.VMEM((K, D), jnp.float32),
        pltpu.VMEM_SHARED((N_PAD, D), jnp.float32),
        pltpu.SemaphoreType.DMA,
    ],
)(_spmm_body)


def _conv_body(x_ref, agg_ref, ws_ref, wself_ref, wnbr_ref, b_ref, out_ref,
               *, norm, resid):
    x = x_ref[...]
    agg = agg_ref[0] + agg_ref[1]
    ws = ws_ref[0] + ws_ref[1]
    aggn = agg / (ws[:, None] + 1e-5)
    y = (jnp.dot(x, wself_ref[...], preferred_element_type=jnp.float32)
         + jnp.dot(aggn, wnbr_ref[...], preferred_element_type=jnp.float32)
         + b_ref[...])
    if norm:
        mu = jnp.mean(y, axis=0, keepdims=True)
        var = jnp.sum((y - mu) * (y - mu), axis=0, keepdims=True) / (N - 1)
        z = (y - mu) / (jnp.sqrt(var) + 1e-5)
        a = jnp.where(z > 0, z, jnp.exp(z) - 1.0)
        y = (a + x) * 0.5 if resid else a
    out_ref[...] = y


def _conv_call(x, agg, ws, wself, wnbr, b, norm, resid):
    body = functools.partial(_conv_body, norm=norm, resid=resid)
    return pl.pallas_call(
        body,
        out_shape=jax.ShapeDtypeStruct((N, D), jnp.float32),
    )(x, agg, ws, wself, wnbr, b)


_IU_NP = np.triu_indices(ED)
_G1_NP = np.zeros((ED, TRI), np.float32)
_G1_NP[_IU_NP[0], np.arange(TRI)] = 1.0
_G2_NP = np.zeros((TRI, ED), np.float32)
_G2_NP[np.arange(TRI), _IU_NP[1]] = 1.0


def kernel(features, vertices, edges, faces, W1m, b1m, W2m, b2m, Wemb, bemb,
           Wself, Wnbr, bconv):
    del faces
    f32 = jnp.float32
    srcp = jnp.pad(edges[0].astype(jnp.int32), (0, E_PAD - E))
    dstp = jnp.pad(edges[1].astype(jnp.int32), (0, E_PAD - E))
    vpad = jnp.pad(vertices.astype(f32), ((0, 0), (0, D - 3)))

    evp = _ev_call(vpad, srcp, dstp)

    w1p = jnp.pad(W1m, ((0, 0), (0, 13), (0, 0)))
    wep = jnp.pad(Wemb, ((0, 0), (0, 13), (0, 0)))
    wall = _wedge_call(evp, w1p, b1m, W2m, b2m, wep, bemb,
                       jnp.asarray(_G1_NP), jnp.asarray(_G2_NP))
    wall_t = wall.T

    wsum_p = _wsum_call(wall, dstp)

    x = features
    for li in range(NL):
        agg_p = _spmm_call(x, srcp, dstp, wall_t[li])
        x = _conv_call(x, agg_p[:, :N], wsum_p[:, :N, li], Wself[li], Wnbr[li],
                       bconv[li].reshape(1, D), norm=(li < NL - 1),
                       resid=(1 <= li <= NL - 2))
    return x


# speedup vs baseline: 6.0451x; 1.2356x over previous
import functools

import numpy as np
import jax
import jax.numpy as jnp
from jax import lax
from jax.experimental import pallas as pl
from jax.experimental.pallas import tpu as pltpu
from jax.experimental.pallas import tpu_sc as plsc

N = 10000
E = 160000
D = 128
MH = 32
ED = 8
TRI = ED * (ED + 1) // 2
NL = 10

NC = 2
NS = 16
NW = NC * NS
K = 128
NCH = 40
EW = K * NCH
E_PAD = NW * EW
N_PAD = 10240
STRIPE = N_PAD // NS
EB = 4096

_MESH = plsc.VectorSubcoreMesh(
    core_axis_name="c", subcore_axis_name="s", num_cores=NC, num_subcores=NS
)


def _ev_body(vert, src_hbm, dst_hbm, ev_out, src_v, dst_v,
             ra0, ra1, rb0, rb1, evbuf, sa0, sa1, sb0, sb1):
    c = lax.axis_index("c")
    s = lax.axis_index("s")
    wid = c * NS + s
    base0 = wid * EW
    pltpu.sync_copy(src_hbm.at[wid], src_v)
    pltpu.sync_copy(dst_hbm.at[wid], dst_v)
    ras = (ra0, ra1)
    rbs = (rb0, rb1)
    sas = (sa0, sa1)
    sbs = (sb0, sb1)

    pltpu.async_copy(vert.at[src_v.at[0]], ra0, sa0)
    pltpu.async_copy(vert.at[dst_v.at[0]], rb0, sb0)

    def outer(go, carry):
        for b in range(2):
            g = go * 2 + b
            nb = 1 - b
            pltpu.make_async_copy(vert.at[src_v.at[g]], ras[b], sas[b]).wait()
            pltpu.make_async_copy(vert.at[dst_v.at[g]], rbs[b], sbs[b]).wait()

            @pl.when(g + 1 < NCH)
            def _():
                pltpu.async_copy(vert.at[src_v.at[g + 1]], ras[nb], sas[nb])
                pltpu.async_copy(vert.at[dst_v.at[g + 1]], rbs[nb], sbs[nb])

            def sub(i, carry2):
                for u in range(4):
                    r = i * 4 + u
                    evbuf[r] = ras[b][r, pl.ds(0, 16)] - rbs[b][r, pl.ds(0, 16)]
                return carry2

            lax.fori_loop(0, K // 4, sub, 0)
            pltpu.sync_copy(evbuf, ev_out.at[pl.ds(base0 + g * K, K)])
        return carry

    lax.fori_loop(0, NCH // 2, outer, 0)


_ev_call = functools.partial(
    pl.kernel,
    out_type=jax.ShapeDtypeStruct((E_PAD, 16), jnp.float32),
    mesh=_MESH,
    scratch_types=[
        pltpu.VMEM((NCH, K), jnp.int32),
        pltpu.VMEM((NCH, K), jnp.int32),
        pltpu.VMEM((K, D), jnp.float32),
        pltpu.VMEM((K, D), jnp.float32),
        pltpu.VMEM((K, D), jnp.float32),
        pltpu.VMEM((K, D), jnp.float32),
        pltpu.VMEM((K, 16), jnp.float32),
        pltpu.SemaphoreType.DMA,
        pltpu.SemaphoreType.DMA,
        pltpu.SemaphoreType.DMA,
        pltpu.SemaphoreType.DMA,
    ],
)(_ev_body)


def _wedge_body(ev_ref, w1_ref, b1_ref, w2_ref, b2_ref, we_ref, be_ref,
                g1_ref, g2_ref, out_ref):
    ev = ev_ref[...]
    pid = pl.program_id(0)
    eidx = lax.broadcasted_iota(jnp.int32, (EB, 1), 0) + pid * EB
    mask = (eidx < E).astype(jnp.float32)
    g1 = g1_ref[...]
    g2 = g2_ref[...]
    cols = []
    for li in range(NL):
        h = jnp.maximum(
            jnp.dot(ev, w1_ref[li], preferred_element_type=jnp.float32)
            + b1_ref[li], 0.0)
        m = jnp.dot(h, w2_ref[li], preferred_element_type=jnp.float32) + b2_ref[li]
        emb = jnp.dot(ev, we_ref[li], preferred_element_type=jnp.float32) + be_ref[li]
        embi = jnp.dot(emb, g1, preferred_element_type=jnp.float32)
        t = jnp.dot(embi * m, g2, preferred_element_type=jnp.float32)
        q = jnp.sum(t * t, axis=1, keepdims=True)
        cols.append(jnp.exp(-q) * mask)
    cols.append(jnp.zeros((EB, D - NL), jnp.float32))
    out_ref[...] = jnp.concatenate(cols, axis=1)


def _wedge_call(evp, w1p, b1, w2, b2, wep, be, g1, g2):
    n_blocks = E_PAD // EB
    zero3 = lambda i: (0, 0, 0)
    zero2 = lambda i: (0, 0)
    return pl.pallas_call(
        _wedge_body,
        grid=(n_blocks,),
        in_specs=[
            pl.BlockSpec((EB, 16), lambda i: (i, 0)),
            pl.BlockSpec((NL, 16, MH), zero3),
            pl.BlockSpec((NL, MH), zero2),
            pl.BlockSpec((NL, MH, TRI), zero3),
            pl.BlockSpec((NL, TRI), zero2),
            pl.BlockSpec((NL, 16, ED), zero3),
            pl.BlockSpec((NL, ED), zero2),
            pl.BlockSpec((ED, TRI), zero2),
            pl.BlockSpec((TRI, ED), zero2),
        ],
        out_specs=pl.BlockSpec((EB, D), lambda i: (i, 0)),
        out_shape=jax.ShapeDtypeStruct((E_PAD, D), jnp.float32),
    )(evp, w1p, b1, w2, b2, wep, be, g1, g2)


def _wsum_body(w_hbm, dst_hbm, ws_out, dst_v, w0, w1, ws_s, sg0, sg1):
    c = lax.axis_index("c")
    s = lax.axis_index("s")
    wid = c * NS + s
    base0 = wid * EW
    bufs = (w0, w1)
    sems = (sg0, sg1)

    def zrows(i, carry):
        for j in range(D // 16):
            w0[i, pl.ds(j * 16, 16)] = jnp.zeros((16,), jnp.float32)
        return carry

    lax.fori_loop(0, K, zrows, 0)
    for j in range(STRIPE // K):
        pltpu.sync_copy(w0, ws_s.at[pl.ds(s * STRIPE + j * K, K)])
    plsc.subcore_barrier()

    pltpu.sync_copy(dst_hbm.at[wid], dst_v)
    pltpu.async_copy(w_hbm.at[pl.ds(base0, K)], w0, sg0)

    def outer(go, carry):
        for b in range(2):
            g = go * 2 + b
            nb = 1 - b
            pltpu.make_async_copy(
                w_hbm.at[pl.ds(base0 + g * K, K)], bufs[b], sems[b]).wait()

            @pl.when(g + 1 < NCH)
            def _():
                pltpu.async_copy(
                    w_hbm.at[pl.ds(base0 + (g + 1) * K, K)], bufs[nb], sems[nb])

            pltpu.sync_copy(bufs[b], ws_s.at[dst_v.at[g]], add=True)
        return carry

    lax.fori_loop(0, NCH // 2, outer, 0)
    plsc.subcore_barrier()
    for j in range(STRIPE // K):
        r0 = s * STRIPE + j * K
        pltpu.sync_copy(ws_s.at[pl.ds(r0, K)], ws_out.at[c, pl.ds(r0, K)])


_wsum_call = functools.partial(
    pl.kernel,
    out_type=jax.ShapeDtypeStruct((NC, N_PAD, D), jnp.float32),
    mesh=_MESH,
    scratch_types=[
        pltpu.VMEM((NCH, K), jnp.int32),
        pltpu.VMEM((K, D), jnp.float32),
        pltpu.VMEM((K, D), jnp.float32),
        pltpu.VMEM_SHARED((N_PAD, D), jnp.float32),
        pltpu.SemaphoreType.DMA,
        pltpu.SemaphoreType.DMA,
    ],
)(_wsum_body)


def _spmm_body(x_hbm, src_hbm, dst_hbm, w_hbm, agg_out,
               src_v, dst_v, w_v, rows0, rows1, agg_s, sg0, sg1):
    c = lax.axis_index("c")
    s = lax.axis_index("s")
    wid = c * NS + s
    bufs = (rows0, rows1)
    sems = (sg0, sg1)

    def zrows(i, carry):
        for j in range(D // 16):
            rows0[i, pl.ds(j * 16, 16)] = jnp.zeros((16,), jnp.float32)
        return carry

    lax.fori_loop(0, K, zrows, 0)
    for j in range(STRIPE // K):
        pltpu.sync_copy(rows0, agg_s.at[pl.ds(s * STRIPE + j * K, K)])
    plsc.subcore_barrier()

    pltpu.sync_copy(src_hbm.at[wid], src_v)
    pltpu.sync_copy(dst_hbm.at[wid], dst_v)
    pltpu.sync_copy(w_hbm.at[wid], w_v)
    pltpu.async_copy(x_hbm.at[src_v.at[0]], rows0, sg0)

    def outer(go, carry):
        for b in range(2):
            g = go * 2 + b
            nb = 1 - b
            pltpu.make_async_copy(x_hbm.at[src_v.at[g]], bufs[b], sems[b]).wait()

            @pl.when(g + 1 < NCH)
            def _():
                pltpu.async_copy(x_hbm.at[src_v.at[g + 1]], bufs[nb], sems[nb])

            def scale(g2, carry2):
                wgrp = w_v[g, pl.ds(g2 * 16, 16)]
                for l in range(16):
                    wspl = wgrp.at[jnp.full((16,), l, jnp.int32)].get(
                        mode="promise_in_bounds")
                    i = g2 * 16 + l
                    for j in range(D // 16):
                        sl = pl.ds(j * 16, 16)
                        bufs[b][i, sl] = bufs[b][i, sl] * wspl
                return carry2

            lax.fori_loop(0, K // 16, scale, 0)
            pltpu.sync_copy(bufs[b], agg_s.at[dst_v.at[g]], add=True)
        return carry

    lax.fori_loop(0, NCH // 2, outer, 0)
    plsc.subcore_barrier()

    for j in range(STRIPE // K):
        r0 = s * STRIPE + j * K
        pltpu.sync_copy(agg_s.at[pl.ds(r0, K)], agg_out.at[c, pl.ds(r0, K)])


_spmm_call = functools.partial(
    pl.kernel,
    out_type=jax.ShapeDtypeStruct((NC, N_PAD, D), jnp.float32),
    mesh=_MESH,
    scratch_types=[
        pltpu.VMEM((NCH, K), jnp.int32),
        pltpu.VMEM((NCH, K), jnp.int32),
        pltpu.VMEM((NCH, K), jnp.float32),
        pltpu.VMEM((K, D), jnp.float32),
        pltpu.VMEM((K, D), jnp.float32),
        pltpu.VMEM_SHARED((N_PAD, D), jnp.float32),
        pltpu.SemaphoreType.DMA,
        pltpu.SemaphoreType.DMA,
    ],
)(_spmm_body)


def _conv_body(x_ref, agg_ref, ws_ref, wself_ref, wnbr_ref, b_ref, out_ref,
               *, norm, resid):
    x = x_ref[...]
    agg = agg_ref[0] + agg_ref[1]
    ws = ws_ref[0] + ws_ref[1]
    aggn = agg / (ws[:, None] + 1e-5)
    y = (jnp.dot(x, wself_ref[...], preferred_element_type=jnp.float32)
         + jnp.dot(aggn, wnbr_ref[...], preferred_element_type=jnp.float32)
         + b_ref[...])
    if norm:
        mu = jnp.mean(y, axis=0, keepdims=True)
        var = jnp.sum((y - mu) * (y - mu), axis=0, keepdims=True) / (N - 1)
        z = (y - mu) / (jnp.sqrt(var) + 1e-5)
        a = jnp.where(z > 0, z, jnp.exp(z) - 1.0)
        y = (a + x) * 0.5 if resid else a
    out_ref[...] = y


def _conv_call(x, agg, ws, wself, wnbr, b, norm, resid):
    body = functools.partial(_conv_body, norm=norm, resid=resid)
    return pl.pallas_call(
        body,
        out_shape=jax.ShapeDtypeStruct((N, D), jnp.float32),
    )(x, agg, ws, wself, wnbr, b)


_IU_NP = np.triu_indices(ED)
_G1_NP = np.zeros((ED, TRI), np.float32)
_G1_NP[_IU_NP[0], np.arange(TRI)] = 1.0
_G2_NP = np.zeros((TRI, ED), np.float32)
_G2_NP[np.arange(TRI), _IU_NP[1]] = 1.0


def kernel(features, vertices, edges, faces, W1m, b1m, W2m, b2m, Wemb, bemb,
           Wself, Wnbr, bconv):
    del faces
    f32 = jnp.float32
    srcp = jnp.pad(edges[0].astype(jnp.int32), (0, E_PAD - E))
    dstp = jnp.pad(edges[1].astype(jnp.int32), (0, E_PAD - E))
    src3 = srcp.reshape(NW, NCH, K)
    dst3 = dstp.reshape(NW, NCH, K)
    vpad = jnp.pad(vertices.astype(f32), ((0, 0), (0, D - 3)))

    evp = _ev_call(vpad, src3, dst3)

    w1p = jnp.pad(W1m, ((0, 0), (0, 13), (0, 0)))
    wep = jnp.pad(Wemb, ((0, 0), (0, 13), (0, 0)))
    wall = _wedge_call(evp, w1p, b1m, W2m, b2m, wep, bemb,
                       jnp.asarray(_G1_NP), jnp.asarray(_G2_NP))
    wall_t = wall.T

    wsum_p = _wsum_call(wall, dst3)

    x = features
    for li in range(NL):
        agg_p = _spmm_call(x, src3, dst3, wall_t[li].reshape(NW, NCH, K))
        x = _conv_call(x, agg_p[:, :N], wsum_p[:, :N, li], Wself[li], Wnbr[li],
                       bconv[li].reshape(1, D), norm=(li < NL - 1),
                       resid=(1 <= li <= NL - 2))
    return x


# speedup vs baseline: 6.8594x; 1.1347x over previous
import functools

import numpy as np
import jax
import jax.numpy as jnp
from jax import lax
from jax.experimental import pallas as pl
from jax.experimental.pallas import tpu as pltpu
from jax.experimental.pallas import tpu_sc as plsc

N = 10000
E = 160000
D = 128
MH = 32
ED = 8
TRI = ED * (ED + 1) // 2
NL = 10

NC = 2
NS = 16
NW = NC * NS
K = 64
NCH = 80
EW = K * NCH
E_PAD = NW * EW
N_PAD = 10240
STRIPE = N_PAD // NS
EB = 4096

_MESH = plsc.VectorSubcoreMesh(
    core_axis_name="c", subcore_axis_name="s", num_cores=NC, num_subcores=NS
)


def _ev_body(vert, src_hbm, dst_hbm, ev_out, src_v, dst_v,
             ra0, ra1, rb0, rb1, evbuf, sa0, sa1, sb0, sb1):
    c = lax.axis_index("c")
    s = lax.axis_index("s")
    wid = c * NS + s
    base0 = wid * EW
    pltpu.sync_copy(src_hbm.at[wid], src_v)
    pltpu.sync_copy(dst_hbm.at[wid], dst_v)
    ras = (ra0, ra1)
    rbs = (rb0, rb1)
    sas = (sa0, sa1)
    sbs = (sb0, sb1)

    pltpu.async_copy(vert.at[src_v.at[0]], ra0, sa0)
    pltpu.async_copy(vert.at[dst_v.at[0]], rb0, sb0)

    def outer(go, carry):
        for b in range(2):
            g = go * 2 + b
            nb = 1 - b
            pltpu.make_async_copy(vert.at[src_v.at[g]], ras[b], sas[b]).wait()
            pltpu.make_async_copy(vert.at[dst_v.at[g]], rbs[b], sbs[b]).wait()

            @pl.when(g + 1 < NCH)
            def _():
                pltpu.async_copy(vert.at[src_v.at[g + 1]], ras[nb], sas[nb])
                pltpu.async_copy(vert.at[dst_v.at[g + 1]], rbs[nb], sbs[nb])

            def sub(i, carry2):
                for u in range(4):
                    r = i * 4 + u
                    evbuf[r] = ras[b][r, pl.ds(0, 16)] - rbs[b][r, pl.ds(0, 16)]
                return carry2

            lax.fori_loop(0, K // 4, sub, 0)
            pltpu.sync_copy(evbuf, ev_out.at[pl.ds(base0 + g * K, K)])
        return carry

    lax.fori_loop(0, NCH // 2, outer, 0)


_ev_call = functools.partial(
    pl.kernel,
    out_type=jax.ShapeDtypeStruct((E_PAD, 16), jnp.float32),
    mesh=_MESH,
    compiler_params=pltpu.CompilerParams(use_tc_tiling_on_sc=False),
    scratch_types=[
        pltpu.VMEM((NCH, K), jnp.int32),
        pltpu.VMEM((NCH, K), jnp.int32),
        pltpu.VMEM((K, 16), jnp.float32),
        pltpu.VMEM((K, 16), jnp.float32),
        pltpu.VMEM((K, 16), jnp.float32),
        pltpu.VMEM((K, 16), jnp.float32),
        pltpu.VMEM((K, 16), jnp.float32),
        pltpu.SemaphoreType.DMA,
        pltpu.SemaphoreType.DMA,
        pltpu.SemaphoreType.DMA,
        pltpu.SemaphoreType.DMA,
    ],
)(_ev_body)


def _wedge_body(ev_ref, w1_ref, b1_ref, w2_ref, b2_ref, we_ref, be_ref,
                g1_ref, g2_ref, out_ref):
    ev = ev_ref[...]
    pid = pl.program_id(0)
    eidx = lax.broadcasted_iota(jnp.int32, (EB, 1), 0) + pid * EB
    mask = (eidx < E).astype(jnp.float32)
    g1 = g1_ref[...]
    g2 = g2_ref[...]
    cols = []
    for li in range(NL):
        h = jnp.maximum(
            jnp.dot(ev, w1_ref[li], preferred_element_type=jnp.float32)
            + b1_ref[li], 0.0)
        m = jnp.dot(h, w2_ref[li], preferred_element_type=jnp.float32) + b2_ref[li]
        emb = jnp.dot(ev, we_ref[li], preferred_element_type=jnp.float32) + be_ref[li]
        embi = jnp.dot(emb, g1, preferred_element_type=jnp.float32)
        t = jnp.dot(embi * m, g2, preferred_element_type=jnp.float32)
        q = jnp.sum(t * t, axis=1, keepdims=True)
        cols.append(jnp.exp(-q) * mask)
    cols.append(jnp.zeros((EB, 16 - NL), jnp.float32))
    out_ref[...] = jnp.concatenate(cols, axis=1)


def _wedge_call(evp, w1p, b1, w2, b2, wep, be, g1, g2):
    n_blocks = E_PAD // EB
    zero3 = lambda i: (0, 0, 0)
    zero2 = lambda i: (0, 0)
    return pl.pallas_call(
        _wedge_body,
        grid=(n_blocks,),
        in_specs=[
            pl.BlockSpec((EB, 16), lambda i: (i, 0)),
            pl.BlockSpec((NL, 16, MH), zero3),
            pl.BlockSpec((NL, MH), zero2),
            pl.BlockSpec((NL, MH, TRI), zero3),
            pl.BlockSpec((NL, TRI), zero2),
            pl.BlockSpec((NL, 16, ED), zero3),
            pl.BlockSpec((NL, ED), zero2),
            pl.BlockSpec((ED, TRI), zero2),
            pl.BlockSpec((TRI, ED), zero2),
        ],
        out_specs=pl.BlockSpec((EB, 16), lambda i: (i, 0)),
        out_shape=jax.ShapeDtypeStruct((E_PAD, 16), jnp.float32),
    )(evp, w1p, b1, w2, b2, wep, be, g1, g2)


def _wsum_body(w_hbm, dst_hbm, ws_out, dst_v, w0, w1, ws_s, sg0, sg1):
    c = lax.axis_index("c")
    s = lax.axis_index("s")
    wid = c * NS + s
    base0 = wid * EW
    bufs = (w0, w1)
    sems = (sg0, sg1)

    def zrows(i, carry):
        w0[i] = jnp.zeros((16,), jnp.float32)
        return carry

    lax.fori_loop(0, K, zrows, 0)
    for j in range(STRIPE // K):
        pltpu.sync_copy(w0, ws_s.at[pl.ds(s * STRIPE + j * K, K)])
    plsc.subcore_barrier()

    pltpu.sync_copy(dst_hbm.at[wid], dst_v)
    pltpu.async_copy(w_hbm.at[pl.ds(base0, K)], w0, sg0)

    def outer(go, carry):
        for b in range(2):
            g = go * 2 + b
            nb = 1 - b
            pltpu.make_async_copy(
                w_hbm.at[pl.ds(base0 + g * K, K)], bufs[b], sems[b]).wait()

            @pl.when(g + 1 < NCH)
            def _():
                pltpu.async_copy(
                    w_hbm.at[pl.ds(base0 + (g + 1) * K, K)], bufs[nb], sems[nb])

            pltpu.sync_copy(bufs[b], ws_s.at[dst_v.at[g]], add=True)
        return carry

    lax.fori_loop(0, NCH // 2, outer, 0)
    plsc.subcore_barrier()
    for j in range(STRIPE // K):
        r0 = s * STRIPE + j * K
        pltpu.sync_copy(ws_s.at[pl.ds(r0, K)], ws_out.at[c, pl.ds(r0, K)])


_wsum_call = functools.partial(
    pl.kernel,
    out_type=jax.ShapeDtypeStruct((NC, N_PAD, 16), jnp.float32),
    mesh=_MESH,
    compiler_params=pltpu.CompilerParams(use_tc_tiling_on_sc=False),
    scratch_types=[
        pltpu.VMEM((NCH, K), jnp.int32),
        pltpu.VMEM((K, 16), jnp.float32),
        pltpu.VMEM((K, 16), jnp.float32),
        pltpu.VMEM_SHARED((N_PAD, 16), jnp.float32),
        pltpu.SemaphoreType.DMA,
        pltpu.SemaphoreType.DMA,
    ],
)(_wsum_body)


def _spmm_body(x_hbm, src_hbm, dst_hbm, w_hbm, agg_out,
               src_v, dst_v, w_v, rows0, rows1, rows2, agg_s,
               sg0, sg1, sg2, ss0, ss1, ss2):
    c = lax.axis_index("c")
    s = lax.axis_index("s")
    wid = c * NS + s
    bufs = (rows0, rows1, rows2)
    gsems = (sg0, sg1, sg2)
    ssems = (ss0, ss1, ss2)

    def zrows(i, carry):
        for j in range(D // 16):
            rows0[i, pl.ds(j * 16, 16)] = jnp.zeros((16,), jnp.float32)
        return carry

    lax.fori_loop(0, K, zrows, 0)
    for j in range(STRIPE // K):
        pltpu.sync_copy(rows0, agg_s.at[pl.ds(s * STRIPE + j * K, K)])
    plsc.subcore_barrier()

    pltpu.sync_copy(src_hbm.at[wid], src_v)
    pltpu.sync_copy(dst_hbm.at[wid], dst_v)
    pltpu.sync_copy(w_hbm.at[wid], w_v)
    pltpu.async_copy(x_hbm.at[src_v.at[pl.ds(0, K)]], rows0, sg0)

    def step(g, b, first, last):
        bn = (b + 1) % 3
        pltpu.make_async_copy(
            x_hbm.at[src_v.at[pl.ds(g * K, K)]], bufs[b], gsems[b]).wait()

        if not first:
            @pl.when(g >= 2)
            def _():
                pltpu.make_async_copy(
                    bufs[bn], agg_s.at[dst_v.at[g - 2]], ssems[bn]).wait()

        if not last:
            @pl.when(g + 1 < NCH)
            def _():
                pltpu.async_copy(
                    x_hbm.at[src_v.at[pl.ds((g + 1) * K, K)]], bufs[bn],
                    gsems[bn])

        def scale(g2, carry2):
            wgrp = w_v[pl.ds(g * K + g2 * 16, 16)]
            for l in range(16):
                wspl = wgrp.at[jnp.full((16,), l, jnp.int32)].get(
                    mode="promise_in_bounds")
                i = g2 * 16 + l
                for j in range(D // 16):
                    sl = pl.ds(j * 16, 16)
                    bufs[b][i, sl] = bufs[b][i, sl] * wspl
            return carry2

        lax.fori_loop(0, K // 16, scale, 0)
        pltpu.async_copy(bufs[b], agg_s.at[dst_v.at[g]], ssems[b], add=True)

    def outer(go, carry):
        for b in range(3):
            step(go * 3 + b, b, first=False, last=False)
        return carry

    lax.fori_loop(0, (NCH - 2) // 3, outer, 0)
    step(NCH - 2, (NCH - 2) % 3, first=False, last=False)
    step(NCH - 1, (NCH - 1) % 3, first=False, last=True)
    for g in range(NCH - 2, NCH):
        pltpu.make_async_copy(
            bufs[g % 3], agg_s.at[dst_v.at[g]], ssems[g % 3]).wait()
    plsc.subcore_barrier()

    for j in range(STRIPE // K):
        r0 = s * STRIPE + j * K
        pltpu.sync_copy(agg_s.at[pl.ds(r0, K)], agg_out.at[c, pl.ds(r0, K)])


_spmm_call = functools.partial(
    pl.kernel,
    out_type=jax.ShapeDtypeStruct((NC, N_PAD, D), jnp.float32),
    mesh=_MESH,
    scratch_types=[
        pltpu.VMEM((EW,), jnp.int32),
        pltpu.VMEM((NCH, K), jnp.int32),
        pltpu.VMEM((EW,), jnp.float32),
        pltpu.VMEM((K, D), jnp.float32),
        pltpu.VMEM((K, D), jnp.float32),
        pltpu.VMEM((K, D), jnp.float32),
        pltpu.VMEM_SHARED((N_PAD, D), jnp.float32),
        pltpu.SemaphoreType.DMA,
        pltpu.SemaphoreType.DMA,
        pltpu.SemaphoreType.DMA,
        pltpu.SemaphoreType.DMA,
        pltpu.SemaphoreType.DMA,
        pltpu.SemaphoreType.DMA,
    ],
)(_spmm_body)


def _conv_body(x_ref, agg_ref, ws_ref, wself_ref, wnbr_ref, b_ref, out_ref,
               *, norm, resid):
    x = x_ref[...]
    agg = agg_ref[0] + agg_ref[1]
    ws = ws_ref[0] + ws_ref[1]
    aggn = agg / (ws[:, None] + 1e-5)
    y = (jnp.dot(x, wself_ref[...], preferred_element_type=jnp.float32)
         + jnp.dot(aggn, wnbr_ref[...], preferred_element_type=jnp.float32)
         + b_ref[...])
    if norm:
        mu = jnp.mean(y, axis=0, keepdims=True)
        var = jnp.sum((y - mu) * (y - mu), axis=0, keepdims=True) / (N - 1)
        z = (y - mu) / (jnp.sqrt(var) + 1e-5)
        a = jnp.where(z > 0, z, jnp.exp(z) - 1.0)
        y = (a + x) * 0.5 if resid else a
    out_ref[...] = y


def _conv_call(x, agg, ws, wself, wnbr, b, norm, resid):
    body = functools.partial(_conv_body, norm=norm, resid=resid)
    return pl.pallas_call(
        body,
        out_shape=jax.ShapeDtypeStruct((N, D), jnp.float32),
    )(x, agg, ws, wself, wnbr, b)


_IU_NP = np.triu_indices(ED)
_G1_NP = np.zeros((ED, TRI), np.float32)
_G1_NP[_IU_NP[0], np.arange(TRI)] = 1.0
_G2_NP = np.zeros((TRI, ED), np.float32)
_G2_NP[np.arange(TRI), _IU_NP[1]] = 1.0


def kernel(features, vertices, edges, faces, W1m, b1m, W2m, b2m, Wemb, bemb,
           Wself, Wnbr, bconv):
    del faces
    f32 = jnp.float32
    srcp = jnp.pad(edges[0].astype(jnp.int32), (0, E_PAD - E))
    dstp = jnp.pad(edges[1].astype(jnp.int32), (0, E_PAD - E))
    src3 = srcp.reshape(NW, NCH, K)
    dst3 = dstp.reshape(NW, NCH, K)
    src2 = srcp.reshape(NW, EW)
    vpad = jnp.pad(vertices.astype(f32), ((0, 0), (0, 13)))

    evp = _ev_call(vpad, src3, dst3)

    w1p = jnp.pad(W1m, ((0, 0), (0, 13), (0, 0)))
    wep = jnp.pad(Wemb, ((0, 0), (0, 13), (0, 0)))
    wall = _wedge_call(evp, w1p, b1m, W2m, b2m, wep, bemb,
                       jnp.asarray(_G1_NP), jnp.asarray(_G2_NP))
    wall_t = wall.T

    wsum_p = _wsum_call(wall, dst3)

    x = features
    for li in range(NL):
        agg_p = _spmm_call(x, src2, dst3, wall_t[li].reshape(NW, EW))
        x = _conv_call(x, agg_p[:, :N], wsum_p[:, :N, li], Wself[li], Wnbr[li],
                       bconv[li].reshape(1, D), norm=(li < NL - 1),
                       resid=(1 <= li <= NL - 2))
    return x
